# Initial kernel scaffold; baseline (speedup 1.0000x reference)
import jax
import jax.numpy as jnp
from jax.experimental import pallas as pl


def kernel(x, edge_index, ptr, batch, W1, b1, W2, b2, W3, b3):
    raise NotImplementedError("write your pallas kernel here")


# speedup vs baseline: 5.7428x; 5.7428x over previous
import functools

import jax
import jax.numpy as jnp
from jax import lax
from jax.experimental import pallas as pl
from jax.experimental.pallas import tpu as pltpu
from jax.experimental.pallas import tpu_sc as plsc

N, E, D, B = 10000, 160000, 256, 16
NP = 10240
EP = 163840
NACC = 11264
JUNK = 10240
CH = 128
HD = D // 2
SP = NP // 16
SPD = NACC // 16
R = 1024
GRID = NP // R

_f32 = jnp.float32
_mesh = plsc.VectorSubcoreMesh(core_axis_name="c", subcore_axis_name="s")


@functools.partial(
    pl.kernel,
    out_type=jax.ShapeDtypeStruct((2 * NACC, HD), _f32),
    mesh=_mesh,
    scratch_types=[
        pltpu.VMEM_SHARED((NACC, HD), _f32),
        pltpu.VMEM((CH,), jnp.int32),
        pltpu.VMEM((CH, HD), _f32),
    ],
)
def _sc_degree(dst_hbm, ones_hbm, zeros_hbm, cnt_hbm, deg_sh, didx, ones_v):
    c = lax.axis_index("c")
    s = lax.axis_index("s")
    pltpu.sync_copy(zeros_hbm.at[pl.ds(s * SPD, SPD)],
                    deg_sh.at[pl.ds(s * SPD, SPD)])
    pltpu.sync_copy(ones_hbm, ones_v)
    plsc.subcore_barrier()
    base = (c * 16 + s) * (EP // 32)

    @pl.loop(0, EP // 32 // CH)
    def _(k):
        pltpu.sync_copy(dst_hbm.at[pl.ds(base + k * CH, CH)], didx)
        pltpu.sync_copy(ones_v, deg_sh.at[didx], add=True)

    plsc.subcore_barrier()
    pltpu.sync_copy(deg_sh.at[pl.ds(s * SPD, SPD)],
                    cnt_hbm.at[pl.ds(c * NACC + s * SPD, SPD)])


@functools.partial(
    pl.kernel,
    out_type=[jax.ShapeDtypeStruct((NP, HD), _f32),
              jax.ShapeDtypeStruct((NP, HD), _f32)],
    mesh=_mesh,
    scratch_types=[
        pltpu.VMEM_SHARED((NACC, HD), _f32),
        pltpu.VMEM((CH,), jnp.int32),
        pltpu.VMEM((CH,), jnp.int32),
        pltpu.VMEM((CH,), jnp.int32),
        pltpu.VMEM((CH,), jnp.int32),
        pltpu.VMEM((2, CH, HD), _f32),
        pltpu.SemaphoreType.DMA,
        pltpu.SemaphoreType.DMA,
    ],
)
def _sc_scatter(ya_hbm, yb_hbm, src_hbm, dst_hbm, za_hbm, zb_hbm,
                acc, sidx0, sidx1, didx0, didx1, rows, sem0, sem1):
    c = lax.axis_index("c")
    s = lax.axis_index("s")
    ebase = s * (EP // 16)

    def run(y_ref):
        pltpu.sync_copy(y_ref.at[pl.ds(s * SP, SP)], acc.at[pl.ds(s * SP, SP)])
        plsc.subcore_barrier()

        @pl.loop(0, EP // 16 // CH, step=2)
        def _(k):
            pltpu.sync_copy(src_hbm.at[pl.ds(ebase + k * CH, CH)], sidx0)
            pltpu.sync_copy(src_hbm.at[pl.ds(ebase + (k + 1) * CH, CH)], sidx1)
            g0 = pltpu.async_copy(y_ref.at[sidx0], rows.at[0], sem0)
            g1 = pltpu.async_copy(y_ref.at[sidx1], rows.at[1], sem1)
            pltpu.sync_copy(dst_hbm.at[pl.ds(ebase + k * CH, CH)], didx0)
            pltpu.sync_copy(dst_hbm.at[pl.ds(ebase + (k + 1) * CH, CH)], didx1)
            g0.wait()
            pltpu.sync_copy(rows.at[0], acc.at[didx0], add=True)
            g1.wait()
            pltpu.sync_copy(rows.at[1], acc.at[didx1], add=True)

    @pl.when(c == 0)
    def _():
        run(ya_hbm)

    @pl.when(c == 1)
    def _():
        run(yb_hbm)

    plsc.subcore_barrier()

    @pl.when(c == 0)
    def _():
        pltpu.sync_copy(acc.at[pl.ds(s * SP, SP)], za_hbm.at[pl.ds(s * SP, SP)])

    @pl.when(c == 1)
    def _():
        pltpu.sync_copy(acc.at[pl.ds(s * SP, SP)], zb_hbm.at[pl.ds(s * SP, SP)])


def _prep_body(c0_ref, c1_ref, x_ref, w_ref, ya_ref, yb_ref, dinv_ref):
    deg = c0_ref[:, 0:1] + c1_ref[:, 0:1] + 1.0
    dinv = lax.rsqrt(deg)
    y = jnp.dot(x_ref[...], w_ref[...], preferred_element_type=_f32) * dinv
    ya_ref[...] = y[:, :HD]
    yb_ref[...] = y[:, HD:]
    dinv_ref[...] = dinv


_tc_prep = pl.pallas_call(
    _prep_body,
    grid=(GRID,),
    in_specs=[
        pl.BlockSpec((R, HD), lambda i: (i, 0)),
        pl.BlockSpec((R, HD), lambda i: (NACC // R + i, 0)),
        pl.BlockSpec((R, D), lambda i: (i, 0)),
        pl.BlockSpec((D, D), lambda i: (0, 0)),
    ],
    out_specs=[
        pl.BlockSpec((R, HD), lambda i: (i, 0)),
        pl.BlockSpec((R, HD), lambda i: (i, 0)),
        pl.BlockSpec((R, 1), lambda i: (i, 0)),
    ],
    out_shape=[
        jax.ShapeDtypeStruct((NP, HD), _f32),
        jax.ShapeDtypeStruct((NP, HD), _f32),
        jax.ShapeDtypeStruct((NP, 1), _f32),
    ],
)


def _mid_body(za_ref, zb_ref, dinv_ref, b_ref, w_ref, ya_ref, yb_ref):
    z = jnp.concatenate([za_ref[...], zb_ref[...]], axis=1)
    dinv = dinv_ref[...]
    h = jnp.maximum(z * dinv + b_ref[...], 0.0)
    y = jnp.dot(h, w_ref[...], preferred_element_type=_f32) * dinv
    ya_ref[...] = y[:, :HD]
    yb_ref[...] = y[:, HD:]


_tc_mid = pl.pallas_call(
    _mid_body,
    grid=(GRID,),
    in_specs=[
        pl.BlockSpec((R, HD), lambda i: (i, 0)),
        pl.BlockSpec((R, HD), lambda i: (i, 0)),
        pl.BlockSpec((R, 1), lambda i: (i, 0)),
        pl.BlockSpec((1, D), lambda i: (0, 0)),
        pl.BlockSpec((D, D), lambda i: (0, 0)),
    ],
    out_specs=[
        pl.BlockSpec((R, HD), lambda i: (i, 0)),
        pl.BlockSpec((R, HD), lambda i: (i, 0)),
    ],
    out_shape=[
        jax.ShapeDtypeStruct((NP, HD), _f32),
        jax.ShapeDtypeStruct((NP, HD), _f32),
    ],
)


def _final_body(za_ref, zb_ref, dinv_ref, b_ref, bat_ref, out_ref,
                sums_scr, cnts_scr):
    i = pl.program_id(0)
    z = jnp.concatenate([za_ref[...], zb_ref[...]], axis=1)
    h = z * dinv_ref[...] + b_ref[...]
    bat = bat_ref[0, 0, :]
    onehot = (bat[:, None]
              == lax.broadcasted_iota(jnp.int32, (R, B), 1)).astype(_f32)
    blk_sums = lax.dot_general(onehot, h, (((0,), (0,)), ((), ())),
                               preferred_element_type=_f32)
    blk_cnts = jnp.broadcast_to(jnp.sum(onehot, axis=0)[:, None], (B, HD))

    @pl.when(i == 0)
    def _():
        sums_scr[...] = blk_sums
        cnts_scr[...] = blk_cnts

    @pl.when(i > 0)
    def _():
        sums_scr[...] += blk_sums
        cnts_scr[...] += blk_cnts

    @pl.when(i == GRID - 1)
    def _():
        out_ref[...] = sums_scr[...] / jnp.maximum(cnts_scr[:, 0:1], 1.0)


_tc_final = pl.pallas_call(
    _final_body,
    grid=(GRID,),
    in_specs=[
        pl.BlockSpec((R, HD), lambda i: (i, 0)),
        pl.BlockSpec((R, HD), lambda i: (i, 0)),
        pl.BlockSpec((R, 1), lambda i: (i, 0)),
        pl.BlockSpec((1, D), lambda i: (0, 0)),
        pl.BlockSpec((1, 1, R), lambda i: (i, 0, 0)),
    ],
    out_specs=pl.BlockSpec((B, D), lambda i: (0, 0)),
    out_shape=jax.ShapeDtypeStruct((B, D), _f32),
    scratch_shapes=[
        pltpu.VMEM((B, D), _f32),
        pltpu.VMEM((B, HD), _f32),
    ],
)


def kernel(x, edge_index, ptr, batch, W1, b1, W2, b2, W3, b3):
    x = x.astype(_f32)
    xp = jnp.zeros((NP, D), _f32).at[:N].set(x)
    src = edge_index[0]
    dst = edge_index[1]
    srcp = jnp.concatenate([src, jnp.zeros((EP - E,), jnp.int32)])
    dstp = jnp.concatenate([dst, jnp.full((EP - E,), JUNK, jnp.int32)])
    batp = jnp.concatenate(
        [batch, jnp.full((NP - N,), B, jnp.int32)]).reshape(GRID, 1, R)
    onescol = jnp.zeros((CH, HD), _f32).at[:, 0].set(1.0)
    zeros_acc = jnp.zeros((NACC, HD), _f32)

    cnt = _sc_degree(dstp, onescol, zeros_acc)
    ya, yb, dinv = _tc_prep(cnt, cnt, xp, W1)
    za, zb = _sc_scatter(ya, yb, srcp, dstp)
    ya, yb = _tc_mid(za, zb, dinv, b1.reshape(1, D), W2)
    za, zb = _sc_scatter(ya, yb, srcp, dstp)
    ya, yb = _tc_mid(za, zb, dinv, b2.reshape(1, D), W3)
    za, zb = _sc_scatter(ya, yb, srcp, dstp)
    mean = _tc_final(za, zb, dinv, b3.reshape(1, D), batp)
    return mean.reshape(-1)


# speedup vs baseline: 5.9253x; 1.0318x over previous
import functools

import jax
import jax.numpy as jnp
from jax import lax
from jax.experimental import pallas as pl
from jax.experimental.pallas import tpu as pltpu
from jax.experimental.pallas import tpu_sc as plsc

N, E, D, B = 10000, 160000, 256, 16
NP = 10240
EP = 163840
NACC = 11264
JUNK = 10240
CH = 128
HD = D // 2
SP = NP // 16
SPD = NACC // 16
R = 1024
GRID = NP // R

_f32 = jnp.float32
_mesh = plsc.VectorSubcoreMesh(core_axis_name="c", subcore_axis_name="s")


@functools.partial(
    pl.kernel,
    out_type=jax.ShapeDtypeStruct((2 * NACC, HD), _f32),
    mesh=_mesh,
    scratch_types=[
        pltpu.VMEM_SHARED((NACC, HD), _f32),
        pltpu.VMEM((4, CH), jnp.int32),
        pltpu.VMEM((CH, HD), _f32),
        pltpu.SemaphoreType.DMA,
        pltpu.SemaphoreType.DMA,
        pltpu.SemaphoreType.DMA,
        pltpu.SemaphoreType.DMA,
    ],
)
def _sc_degree(dst_hbm, ones_hbm, zeros_hbm, cnt_hbm, deg_sh, didx, ones_v,
               sm0, sm1, sm2, sm3):
    c = lax.axis_index("c")
    s = lax.axis_index("s")
    ssem = (sm0, sm1, sm2, sm3)
    pltpu.sync_copy(zeros_hbm.at[pl.ds(s * SPD, SPD)],
                    deg_sh.at[pl.ds(s * SPD, SPD)])
    pltpu.sync_copy(ones_hbm, ones_v)
    plsc.subcore_barrier()
    cbase = (c * 16 + s) * (EP // 32 // CH)

    @pl.loop(0, EP // 32 // CH, step=4)
    def _(k):
        pltpu.sync_copy(dst_hbm.at[pl.ds(cbase + k, 4)], didx)
        sd = [pltpu.async_copy(ones_v, deg_sh.at[didx.at[j]], ssem[j],
                               add=True) for j in range(4)]
        for j in range(4):
            sd[j].wait()

    plsc.subcore_barrier()
    pltpu.sync_copy(deg_sh.at[pl.ds(s * SPD, SPD)],
                    cnt_hbm.at[pl.ds(c * NACC + s * SPD, SPD)])


@functools.partial(
    pl.kernel,
    out_type=[jax.ShapeDtypeStruct((NP, HD), _f32),
              jax.ShapeDtypeStruct((NP, HD), _f32)],
    mesh=_mesh,
    scratch_types=[
        pltpu.VMEM_SHARED((NACC, HD), _f32),
        pltpu.VMEM((2, CH), jnp.int32),
        pltpu.VMEM((2, CH), jnp.int32),
        pltpu.VMEM((2, CH, HD), _f32),
        pltpu.SemaphoreType.DMA,
        pltpu.SemaphoreType.DMA,
        pltpu.SemaphoreType.DMA,
        pltpu.SemaphoreType.DMA,
    ],
)
def _sc_scatter(ya_hbm, yb_hbm, src_hbm, dst_hbm, za_hbm, zb_hbm,
                acc, sidx, didx, rows, g0, g1, t0, t1):
    c = lax.axis_index("c")
    s = lax.axis_index("s")
    gsem = (g0, g1)
    ssem = (t0, t1)
    cbase = s * (EP // 16 // CH)

    def run(y_ref):
        pltpu.sync_copy(y_ref.at[pl.ds(s * SP, SP)], acc.at[pl.ds(s * SP, SP)])
        plsc.subcore_barrier()

        @pl.loop(0, EP // 16 // CH, step=2)
        def _(k):
            pltpu.sync_copy(src_hbm.at[pl.ds(cbase + k, 2)], sidx)
            pltpu.sync_copy(dst_hbm.at[pl.ds(cbase + k, 2)], didx)
            gd = [pltpu.async_copy(y_ref.at[sidx.at[j]], rows.at[j], gsem[j])
                  for j in range(2)]
            sd = []
            for j in range(2):
                gd[j].wait()
                sd.append(pltpu.async_copy(rows.at[j], acc.at[didx.at[j]],
                                           ssem[j], add=True))
            for j in range(2):
                sd[j].wait()

    @pl.when(c == 0)
    def _():
        run(ya_hbm)

    @pl.when(c == 1)
    def _():
        run(yb_hbm)

    plsc.subcore_barrier()

    @pl.when(c == 0)
    def _():
        pltpu.sync_copy(acc.at[pl.ds(s * SP, SP)], za_hbm.at[pl.ds(s * SP, SP)])

    @pl.when(c == 1)
    def _():
        pltpu.sync_copy(acc.at[pl.ds(s * SP, SP)], zb_hbm.at[pl.ds(s * SP, SP)])


def _prep_body(c0_ref, c1_ref, x_ref, w_ref, ya_ref, yb_ref, dinv_ref):
    deg = c0_ref[:, 0:1] + c1_ref[:, 0:1] + 1.0
    dinv = lax.rsqrt(deg)
    y = jnp.dot(x_ref[...], w_ref[...], preferred_element_type=_f32) * dinv
    ya_ref[...] = y[:, :HD]
    yb_ref[...] = y[:, HD:]
    dinv_ref[...] = dinv


_tc_prep = pl.pallas_call(
    _prep_body,
    grid=(GRID,),
    in_specs=[
        pl.BlockSpec((R, HD), lambda i: (i, 0)),
        pl.BlockSpec((R, HD), lambda i: (NACC // R + i, 0)),
        pl.BlockSpec((R, D), lambda i: (i, 0)),
        pl.BlockSpec((D, D), lambda i: (0, 0)),
    ],
    out_specs=[
        pl.BlockSpec((R, HD), lambda i: (i, 0)),
        pl.BlockSpec((R, HD), lambda i: (i, 0)),
        pl.BlockSpec((R, 1), lambda i: (i, 0)),
    ],
    out_shape=[
        jax.ShapeDtypeStruct((NP, HD), _f32),
        jax.ShapeDtypeStruct((NP, HD), _f32),
        jax.ShapeDtypeStruct((NP, 1), _f32),
    ],
)


def _mid_body(za_ref, zb_ref, dinv_ref, b_ref, w_ref, ya_ref, yb_ref):
    z = jnp.concatenate([za_ref[...], zb_ref[...]], axis=1)
    dinv = dinv_ref[...]
    h = jnp.maximum(z * dinv + b_ref[...], 0.0)
    y = jnp.dot(h, w_ref[...], preferred_element_type=_f32) * dinv
    ya_ref[...] = y[:, :HD]
    yb_ref[...] = y[:, HD:]


_tc_mid = pl.pallas_call(
    _mid_body,
    grid=(GRID,),
    in_specs=[
        pl.BlockSpec((R, HD), lambda i: (i, 0)),
        pl.BlockSpec((R, HD), lambda i: (i, 0)),
        pl.BlockSpec((R, 1), lambda i: (i, 0)),
        pl.BlockSpec((1, D), lambda i: (0, 0)),
        pl.BlockSpec((D, D), lambda i: (0, 0)),
    ],
    out_specs=[
        pl.BlockSpec((R, HD), lambda i: (i, 0)),
        pl.BlockSpec((R, HD), lambda i: (i, 0)),
    ],
    out_shape=[
        jax.ShapeDtypeStruct((NP, HD), _f32),
        jax.ShapeDtypeStruct((NP, HD), _f32),
    ],
)


def _final_body(za_ref, zb_ref, dinv_ref, b_ref, bat_ref, out_ref,
                sums_scr, cnts_scr):
    i = pl.program_id(0)
    z = jnp.concatenate([za_ref[...], zb_ref[...]], axis=1)
    h = z * dinv_ref[...] + b_ref[...]
    bat = bat_ref[0, 0, :]
    onehot = (bat[:, None]
              == lax.broadcasted_iota(jnp.int32, (R, B), 1)).astype(_f32)
    blk_sums = lax.dot_general(onehot, h, (((0,), (0,)), ((), ())),
                               preferred_element_type=_f32)
    blk_cnts = jnp.broadcast_to(jnp.sum(onehot, axis=0)[:, None], (B, HD))

    @pl.when(i == 0)
    def _():
        sums_scr[...] = blk_sums
        cnts_scr[...] = blk_cnts

    @pl.when(i > 0)
    def _():
        sums_scr[...] += blk_sums
        cnts_scr[...] += blk_cnts

    @pl.when(i == GRID - 1)
    def _():
        out_ref[...] = sums_scr[...] / jnp.maximum(cnts_scr[:, 0:1], 1.0)


_tc_final = pl.pallas_call(
    _final_body,
    grid=(GRID,),
    in_specs=[
        pl.BlockSpec((R, HD), lambda i: (i, 0)),
        pl.BlockSpec((R, HD), lambda i: (i, 0)),
        pl.BlockSpec((R, 1), lambda i: (i, 0)),
        pl.BlockSpec((1, D), lambda i: (0, 0)),
        pl.BlockSpec((1, 1, R), lambda i: (i, 0, 0)),
    ],
    out_specs=pl.BlockSpec((B, D), lambda i: (0, 0)),
    out_shape=jax.ShapeDtypeStruct((B, D), _f32),
    scratch_shapes=[
        pltpu.VMEM((B, D), _f32),
        pltpu.VMEM((B, HD), _f32),
    ],
)


def kernel(x, edge_index, ptr, batch, W1, b1, W2, b2, W3, b3):
    x = x.astype(_f32)
    xp = jnp.zeros((NP, D), _f32).at[:N].set(x)
    src = edge_index[0]
    dst = edge_index[1]
    srcp = jnp.concatenate(
        [src, jnp.zeros((EP - E,), jnp.int32)]).reshape(EP // CH, CH)
    dstp = jnp.concatenate(
        [dst, jnp.full((EP - E,), JUNK, jnp.int32)]).reshape(EP // CH, CH)
    batp = jnp.concatenate(
        [batch, jnp.full((NP - N,), B, jnp.int32)]).reshape(GRID, 1, R)
    onescol = jnp.zeros((CH, HD), _f32).at[:, 0].set(1.0)
    zeros_acc = jnp.zeros((NACC, HD), _f32)

    cnt = _sc_degree(dstp, onescol, zeros_acc)
    ya, yb, dinv = _tc_prep(cnt, cnt, xp, W1)
    za, zb = _sc_scatter(ya, yb, srcp, dstp)
    ya, yb = _tc_mid(za, zb, dinv, b1.reshape(1, D), W2)
    za, zb = _sc_scatter(ya, yb, srcp, dstp)
    ya, yb = _tc_mid(za, zb, dinv, b2.reshape(1, D), W3)
    za, zb = _sc_scatter(ya, yb, srcp, dstp)
    mean = _tc_final(za, zb, dinv, b3.reshape(1, D), batp)
    return mean.reshape(-1)


# speedup vs baseline: 6.3607x; 1.0735x over previous
import functools

import jax
import jax.numpy as jnp
from jax import lax
from jax.experimental import pallas as pl
from jax.experimental.pallas import tpu as pltpu
from jax.experimental.pallas import tpu_sc as plsc

N, E, D, B = 10000, 160000, 256, 16
NP = 10240
EP = 163840
NACC = 11264
JUNK = 10240
CH = 128
HD = D // 2
SP = NP // 16
SPD = NACC // 16
R = 1024
GRID = NP // R

_f32 = jnp.float32
_mesh = plsc.VectorSubcoreMesh(core_axis_name="c", subcore_axis_name="s")


@functools.partial(
    pl.kernel,
    out_type=jax.ShapeDtypeStruct((2 * NACC, HD), _f32),
    mesh=_mesh,
    scratch_types=[
        pltpu.VMEM_SHARED((NACC, HD), _f32),
        pltpu.VMEM((EP // 32 // CH, CH), jnp.int32),
        pltpu.VMEM((CH, HD), _f32),
        pltpu.SemaphoreType.DMA,
        pltpu.SemaphoreType.DMA,
        pltpu.SemaphoreType.DMA,
        pltpu.SemaphoreType.DMA,
    ],
)
def _sc_degree(dst_hbm, ones_hbm, zeros_hbm, cnt_hbm, deg_sh, didx, ones_v,
               sm0, sm1, sm2, sm3):
    c = lax.axis_index("c")
    s = lax.axis_index("s")
    ssem = (sm0, sm1, sm2, sm3)
    pltpu.sync_copy(zeros_hbm.at[pl.ds(s * SPD, SPD)],
                    deg_sh.at[pl.ds(s * SPD, SPD)])
    pltpu.sync_copy(ones_hbm, ones_v)
    cbase = (c * 16 + s) * (EP // 32 // CH)
    pltpu.sync_copy(dst_hbm.at[pl.ds(cbase, EP // 32 // CH)], didx)
    plsc.subcore_barrier()

    @pl.loop(0, EP // 32 // CH, step=4)
    def _(k):
        sd = [pltpu.async_copy(ones_v, deg_sh.at[didx.at[k + j]], ssem[j],
                               add=True) for j in range(4)]
        for j in range(4):
            sd[j].wait()

    plsc.subcore_barrier()
    pltpu.sync_copy(deg_sh.at[pl.ds(s * SPD, SPD)],
                    cnt_hbm.at[pl.ds(c * NACC + s * SPD, SPD)])


@functools.partial(
    pl.kernel,
    out_type=[jax.ShapeDtypeStruct((NP, HD), _f32),
              jax.ShapeDtypeStruct((NP, HD), _f32)],
    mesh=_mesh,
    scratch_types=[
        pltpu.VMEM_SHARED((NACC, HD), _f32),
        pltpu.VMEM((16, CH), jnp.int32),
        pltpu.VMEM((16, CH), jnp.int32),
        pltpu.VMEM((2, CH, HD), _f32),
        pltpu.SemaphoreType.DMA,
        pltpu.SemaphoreType.DMA,
        pltpu.SemaphoreType.DMA,
        pltpu.SemaphoreType.DMA,
    ],
)
def _sc_scatter(ya_hbm, yb_hbm, src_hbm, dst_hbm, za_hbm, zb_hbm,
                acc, sidx, didx, rows, g0, g1, t0, t1):
    c = lax.axis_index("c")
    s = lax.axis_index("s")
    gsem = (g0, g1)
    ssem = (t0, t1)
    cbase = s * (EP // 16 // CH)

    def run(y_ref):
        pltpu.sync_copy(y_ref.at[pl.ds(s * SP, SP)], acc.at[pl.ds(s * SP, SP)])
        plsc.subcore_barrier()

        @pl.loop(0, EP // 16 // CH, step=16)
        def _(k):
            pltpu.sync_copy(src_hbm.at[pl.ds(cbase + k, 16)], sidx)
            pltpu.sync_copy(dst_hbm.at[pl.ds(cbase + k, 16)], didx)
            for m in range(8):
                gd = [pltpu.async_copy(y_ref.at[sidx.at[2 * m + j]],
                                       rows.at[j], gsem[j])
                      for j in range(2)]
                sd = []
                for j in range(2):
                    gd[j].wait()
                    sd.append(pltpu.async_copy(rows.at[j],
                                               acc.at[didx.at[2 * m + j]],
                                               ssem[j], add=True))
                for j in range(2):
                    sd[j].wait()

    @pl.when(c == 0)
    def _():
        run(ya_hbm)

    @pl.when(c == 1)
    def _():
        run(yb_hbm)

    plsc.subcore_barrier()

    @pl.when(c == 0)
    def _():
        pltpu.sync_copy(acc.at[pl.ds(s * SP, SP)], za_hbm.at[pl.ds(s * SP, SP)])

    @pl.when(c == 1)
    def _():
        pltpu.sync_copy(acc.at[pl.ds(s * SP, SP)], zb_hbm.at[pl.ds(s * SP, SP)])


def _prep_body(c0_ref, c1_ref, x_ref, w_ref, ya_ref, yb_ref, dinv_ref):
    deg = c0_ref[:, 0:1] + c1_ref[:, 0:1] + 1.0
    dinv = lax.rsqrt(deg)
    y = jnp.dot(x_ref[...], w_ref[...], preferred_element_type=_f32) * dinv
    ya_ref[...] = y[:, :HD]
    yb_ref[...] = y[:, HD:]
    dinv_ref[...] = dinv


_tc_prep = pl.pallas_call(
    _prep_body,
    grid=(GRID,),
    in_specs=[
        pl.BlockSpec((R, HD), lambda i: (i, 0)),
        pl.BlockSpec((R, HD), lambda i: (NACC // R + i, 0)),
        pl.BlockSpec((R, D), lambda i: (i, 0)),
        pl.BlockSpec((D, D), lambda i: (0, 0)),
    ],
    out_specs=[
        pl.BlockSpec((R, HD), lambda i: (i, 0)),
        pl.BlockSpec((R, HD), lambda i: (i, 0)),
        pl.BlockSpec((R, 1), lambda i: (i, 0)),
    ],
    out_shape=[
        jax.ShapeDtypeStruct((NP, HD), _f32),
        jax.ShapeDtypeStruct((NP, HD), _f32),
        jax.ShapeDtypeStruct((NP, 1), _f32),
    ],
)


def _mid_body(za_ref, zb_ref, dinv_ref, b_ref, w_ref, ya_ref, yb_ref):
    z = jnp.concatenate([za_ref[...], zb_ref[...]], axis=1)
    dinv = dinv_ref[...]
    h = jnp.maximum(z * dinv + b_ref[...], 0.0)
    y = jnp.dot(h, w_ref[...], preferred_element_type=_f32) * dinv
    ya_ref[...] = y[:, :HD]
    yb_ref[...] = y[:, HD:]


_tc_mid = pl.pallas_call(
    _mid_body,
    grid=(GRID,),
    in_specs=[
        pl.BlockSpec((R, HD), lambda i: (i, 0)),
        pl.BlockSpec((R, HD), lambda i: (i, 0)),
        pl.BlockSpec((R, 1), lambda i: (i, 0)),
        pl.BlockSpec((1, D), lambda i: (0, 0)),
        pl.BlockSpec((D, D), lambda i: (0, 0)),
    ],
    out_specs=[
        pl.BlockSpec((R, HD), lambda i: (i, 0)),
        pl.BlockSpec((R, HD), lambda i: (i, 0)),
    ],
    out_shape=[
        jax.ShapeDtypeStruct((NP, HD), _f32),
        jax.ShapeDtypeStruct((NP, HD), _f32),
    ],
)


def _final_body(za_ref, zb_ref, dinv_ref, b_ref, bat_ref, out_ref,
                sums_scr, cnts_scr):
    i = pl.program_id(0)
    z = jnp.concatenate([za_ref[...], zb_ref[...]], axis=1)
    h = z * dinv_ref[...] + b_ref[...]
    bat = bat_ref[0, 0, :]
    onehot = (bat[:, None]
              == lax.broadcasted_iota(jnp.int32, (R, B), 1)).astype(_f32)
    blk_sums = lax.dot_general(onehot, h, (((0,), (0,)), ((), ())),
                               preferred_element_type=_f32)
    blk_cnts = jnp.broadcast_to(jnp.sum(onehot, axis=0)[:, None], (B, HD))

    @pl.when(i == 0)
    def _():
        sums_scr[...] = blk_sums
        cnts_scr[...] = blk_cnts

    @pl.when(i > 0)
    def _():
        sums_scr[...] += blk_sums
        cnts_scr[...] += blk_cnts

    @pl.when(i == GRID - 1)
    def _():
        out_ref[...] = sums_scr[...] / jnp.maximum(cnts_scr[:, 0:1], 1.0)


_tc_final = pl.pallas_call(
    _final_body,
    grid=(GRID,),
    in_specs=[
        pl.BlockSpec((R, HD), lambda i: (i, 0)),
        pl.BlockSpec((R, HD), lambda i: (i, 0)),
        pl.BlockSpec((R, 1), lambda i: (i, 0)),
        pl.BlockSpec((1, D), lambda i: (0, 0)),
        pl.BlockSpec((1, 1, R), lambda i: (i, 0, 0)),
    ],
    out_specs=pl.BlockSpec((B, D), lambda i: (0, 0)),
    out_shape=jax.ShapeDtypeStruct((B, D), _f32),
    scratch_shapes=[
        pltpu.VMEM((B, D), _f32),
        pltpu.VMEM((B, HD), _f32),
    ],
)


def kernel(x, edge_index, ptr, batch, W1, b1, W2, b2, W3, b3):
    x = x.astype(_f32)
    xp = jnp.zeros((NP, D), _f32).at[:N].set(x)
    src = edge_index[0]
    dst = edge_index[1]
    srcp = jnp.concatenate(
        [src, jnp.zeros((EP - E,), jnp.int32)]).reshape(EP // CH, CH)
    dstp = jnp.concatenate(
        [dst, jnp.full((EP - E,), JUNK, jnp.int32)]).reshape(EP // CH, CH)
    batp = jnp.concatenate(
        [batch, jnp.full((NP - N,), B, jnp.int32)]).reshape(GRID, 1, R)
    onescol = jnp.zeros((CH, HD), _f32).at[:, 0].set(1.0)
    zeros_acc = jnp.zeros((NACC, HD), _f32)

    cnt = _sc_degree(dstp, onescol, zeros_acc)
    ya, yb, dinv = _tc_prep(cnt, cnt, xp, W1)
    za, zb = _sc_scatter(ya, yb, srcp, dstp)
    ya, yb = _tc_mid(za, zb, dinv, b1.reshape(1, D), W2)
    za, zb = _sc_scatter(ya, yb, srcp, dstp)
    ya, yb = _tc_mid(za, zb, dinv, b2.reshape(1, D), W3)
    za, zb = _sc_scatter(ya, yb, srcp, dstp)
    mean = _tc_final(za, zb, dinv, b3.reshape(1, D), batp)
    return mean.reshape(-1)


# speedup vs baseline: 7.1465x; 1.1235x over previous
import functools

import jax
import jax.numpy as jnp
from jax import lax
from jax.experimental import pallas as pl
from jax.experimental.pallas import tpu as pltpu
from jax.experimental.pallas import tpu_sc as plsc

N, E, D, B = 10000, 160000, 256, 16
NP = 10240
EP = 163840
NACC = 11264
JUNK = 10240
CH = 64
HD = D // 2
SP = NP // 16
SPD = NACC // 16
R = 1024
GRID = NP // R

_f32 = jnp.float32
_mesh = plsc.VectorSubcoreMesh(core_axis_name="c", subcore_axis_name="s")


@functools.partial(
    pl.kernel,
    out_type=jax.ShapeDtypeStruct((2 * NACC, HD), _f32),
    mesh=_mesh,
    scratch_types=[
        pltpu.VMEM_SHARED((NACC, HD), _f32),
        pltpu.VMEM((EP // 32 // CH, CH), jnp.int32),
        pltpu.VMEM((CH, HD), _f32),
        pltpu.SemaphoreType.DMA,
        pltpu.SemaphoreType.DMA,
        pltpu.SemaphoreType.DMA,
        pltpu.SemaphoreType.DMA,
    ],
)
def _sc_degree(dst_hbm, ones_hbm, zeros_hbm, cnt_hbm, deg_sh, didx, ones_v,
               sm0, sm1, sm2, sm3):
    c = lax.axis_index("c")
    s = lax.axis_index("s")
    ssem = (sm0, sm1, sm2, sm3)
    pltpu.sync_copy(zeros_hbm.at[pl.ds(s * SPD, SPD)],
                    deg_sh.at[pl.ds(s * SPD, SPD)])
    pltpu.sync_copy(ones_hbm, ones_v)
    cbase = (c * 16 + s) * (EP // 32 // CH)
    pltpu.sync_copy(dst_hbm.at[pl.ds(cbase, EP // 32 // CH)], didx)
    plsc.subcore_barrier()

    @pl.loop(0, EP // 32 // CH, step=4)
    def _(k):
        sd = [pltpu.async_copy(ones_v, deg_sh.at[didx.at[k + j]], ssem[j],
                               add=True) for j in range(4)]
        for j in range(4):
            sd[j].wait()

    plsc.subcore_barrier()
    pltpu.sync_copy(deg_sh.at[pl.ds(s * SPD, SPD)],
                    cnt_hbm.at[pl.ds(c * NACC + s * SPD, SPD)])


@functools.partial(
    pl.kernel,
    out_type=[jax.ShapeDtypeStruct((NP, HD), _f32),
              jax.ShapeDtypeStruct((NP, HD), _f32)],
    mesh=_mesh,
    scratch_types=[
        pltpu.VMEM_SHARED((NACC, HD), _f32),
        pltpu.VMEM((8, CH), jnp.int32),
        pltpu.VMEM((8, CH), jnp.int32),
        pltpu.VMEM((4, CH, HD), _f32),
        pltpu.SemaphoreType.DMA,
        pltpu.SemaphoreType.DMA,
        pltpu.SemaphoreType.DMA,
        pltpu.SemaphoreType.DMA,
        pltpu.SemaphoreType.DMA,
        pltpu.SemaphoreType.DMA,
        pltpu.SemaphoreType.DMA,
        pltpu.SemaphoreType.DMA,
    ],
)
def _sc_scatter(ya_hbm, yb_hbm, src_hbm, dst_hbm, za_hbm, zb_hbm,
                acc, sidx, didx, rows,
                g0, g1, g2, g3, t0, t1, t2, t3):
    c = lax.axis_index("c")
    s = lax.axis_index("s")
    gsem = (g0, g1, g2, g3)
    ssem = (t0, t1, t2, t3)
    cbase = s * (EP // 16 // CH)
    pair = lambda m: (0, 1) if m % 2 == 0 else (2, 3)
    NB = 4

    def run(y_ref):
        pltpu.sync_copy(y_ref.at[pl.ds(s * SP, SP)], acc.at[pl.ds(s * SP, SP)])
        plsc.subcore_barrier()

        @pl.loop(0, EP // 16 // CH, step=2 * NB)
        def _(k):
            pltpu.sync_copy(src_hbm.at[pl.ds(cbase + k, 2 * NB)], sidx)
            pltpu.sync_copy(dst_hbm.at[pl.ds(cbase + k, 2 * NB)], didx)
            gd, sd = {}, {}
            for j, b in enumerate(pair(0)):
                gd[(0, j)] = pltpu.async_copy(y_ref.at[sidx.at[j]],
                                              rows.at[b], gsem[b])
            for m in range(NB):
                if m + 1 < NB:
                    for j, b in enumerate(pair(m + 1)):
                        if m >= 1:
                            sd[(m - 1, j)].wait()
                        gd[(m + 1, j)] = pltpu.async_copy(
                            y_ref.at[sidx.at[2 * (m + 1) + j]],
                            rows.at[b], gsem[b])
                for j, b in enumerate(pair(m)):
                    gd[(m, j)].wait()
                    sd[(m, j)] = pltpu.async_copy(
                        rows.at[b], acc.at[didx.at[2 * m + j]],
                        ssem[b], add=True)
            for m in (NB - 2, NB - 1):
                for j in range(2):
                    sd[(m, j)].wait()

    @pl.when(c == 0)
    def _():
        run(ya_hbm)

    @pl.when(c == 1)
    def _():
        run(yb_hbm)

    plsc.subcore_barrier()

    @pl.when(c == 0)
    def _():
        pltpu.sync_copy(acc.at[pl.ds(s * SP, SP)], za_hbm.at[pl.ds(s * SP, SP)])

    @pl.when(c == 1)
    def _():
        pltpu.sync_copy(acc.at[pl.ds(s * SP, SP)], zb_hbm.at[pl.ds(s * SP, SP)])


def _prep_body(c0_ref, c1_ref, x_ref, w_ref, ya_ref, yb_ref, dinv_ref):
    deg = c0_ref[:, 0:1] + c1_ref[:, 0:1] + 1.0
    dinv = lax.rsqrt(deg)
    y = jnp.dot(x_ref[...], w_ref[...], preferred_element_type=_f32) * dinv
    ya_ref[...] = y[:, :HD]
    yb_ref[...] = y[:, HD:]
    dinv_ref[...] = dinv


_tc_prep = pl.pallas_call(
    _prep_body,
    grid=(GRID,),
    in_specs=[
        pl.BlockSpec((R, HD), lambda i: (i, 0)),
        pl.BlockSpec((R, HD), lambda i: (NACC // R + i, 0)),
        pl.BlockSpec((R, D), lambda i: (i, 0)),
        pl.BlockSpec((D, D), lambda i: (0, 0)),
    ],
    out_specs=[
        pl.BlockSpec((R, HD), lambda i: (i, 0)),
        pl.BlockSpec((R, HD), lambda i: (i, 0)),
        pl.BlockSpec((R, 1), lambda i: (i, 0)),
    ],
    out_shape=[
        jax.ShapeDtypeStruct((NP, HD), _f32),
        jax.ShapeDtypeStruct((NP, HD), _f32),
        jax.ShapeDtypeStruct((NP, 1), _f32),
    ],
)


def _mid_body(za_ref, zb_ref, dinv_ref, b_ref, w_ref, ya_ref, yb_ref):
    z = jnp.concatenate([za_ref[...], zb_ref[...]], axis=1)
    dinv = dinv_ref[...]
    h = jnp.maximum(z * dinv + b_ref[...], 0.0)
    y = jnp.dot(h, w_ref[...], preferred_element_type=_f32) * dinv
    ya_ref[...] = y[:, :HD]
    yb_ref[...] = y[:, HD:]


_tc_mid = pl.pallas_call(
    _mid_body,
    grid=(GRID,),
    in_specs=[
        pl.BlockSpec((R, HD), lambda i: (i, 0)),
        pl.BlockSpec((R, HD), lambda i: (i, 0)),
        pl.BlockSpec((R, 1), lambda i: (i, 0)),
        pl.BlockSpec((1, D), lambda i: (0, 0)),
        pl.BlockSpec((D, D), lambda i: (0, 0)),
    ],
    out_specs=[
        pl.BlockSpec((R, HD), lambda i: (i, 0)),
        pl.BlockSpec((R, HD), lambda i: (i, 0)),
    ],
    out_shape=[
        jax.ShapeDtypeStruct((NP, HD), _f32),
        jax.ShapeDtypeStruct((NP, HD), _f32),
    ],
)


def _final_body(za_ref, zb_ref, dinv_ref, b_ref, bat_ref, out_ref,
                sums_scr, cnts_scr):
    i = pl.program_id(0)
    z = jnp.concatenate([za_ref[...], zb_ref[...]], axis=1)
    h = z * dinv_ref[...] + b_ref[...]
    bat = bat_ref[0, 0, :]
    onehot = (bat[:, None]
              == lax.broadcasted_iota(jnp.int32, (R, B), 1)).astype(_f32)
    blk_sums = lax.dot_general(onehot, h, (((0,), (0,)), ((), ())),
                               preferred_element_type=_f32)
    blk_cnts = jnp.broadcast_to(jnp.sum(onehot, axis=0)[:, None], (B, HD))

    @pl.when(i == 0)
    def _():
        sums_scr[...] = blk_sums
        cnts_scr[...] = blk_cnts

    @pl.when(i > 0)
    def _():
        sums_scr[...] += blk_sums
        cnts_scr[...] += blk_cnts

    @pl.when(i == GRID - 1)
    def _():
        out_ref[...] = sums_scr[...] / jnp.maximum(cnts_scr[:, 0:1], 1.0)


_tc_final = pl.pallas_call(
    _final_body,
    grid=(GRID,),
    in_specs=[
        pl.BlockSpec((R, HD), lambda i: (i, 0)),
        pl.BlockSpec((R, HD), lambda i: (i, 0)),
        pl.BlockSpec((R, 1), lambda i: (i, 0)),
        pl.BlockSpec((1, D), lambda i: (0, 0)),
        pl.BlockSpec((1, 1, R), lambda i: (i, 0, 0)),
    ],
    out_specs=pl.BlockSpec((B, D), lambda i: (0, 0)),
    out_shape=jax.ShapeDtypeStruct((B, D), _f32),
    scratch_shapes=[
        pltpu.VMEM((B, D), _f32),
        pltpu.VMEM((B, HD), _f32),
    ],
)


def kernel(x, edge_index, ptr, batch, W1, b1, W2, b2, W3, b3):
    x = x.astype(_f32)
    xp = jnp.zeros((NP, D), _f32).at[:N].set(x)
    src = edge_index[0]
    dst = edge_index[1]
    srcp = jnp.concatenate(
        [src, jnp.zeros((EP - E,), jnp.int32)]).reshape(EP // CH, CH)
    dstp = jnp.concatenate(
        [dst, jnp.full((EP - E,), JUNK, jnp.int32)]).reshape(EP // CH, CH)
    batp = jnp.concatenate(
        [batch, jnp.full((NP - N,), B, jnp.int32)]).reshape(GRID, 1, R)
    onescol = jnp.zeros((CH, HD), _f32).at[:, 0].set(1.0)
    zeros_acc = jnp.zeros((NACC, HD), _f32)

    cnt = _sc_degree(dstp, onescol, zeros_acc)
    ya, yb, dinv = _tc_prep(cnt, cnt, xp, W1)
    za, zb = _sc_scatter(ya, yb, srcp, dstp)
    ya, yb = _tc_mid(za, zb, dinv, b1.reshape(1, D), W2)
    za, zb = _sc_scatter(ya, yb, srcp, dstp)
    ya, yb = _tc_mid(za, zb, dinv, b2.reshape(1, D), W3)
    za, zb = _sc_scatter(ya, yb, srcp, dstp)
    mean = _tc_final(za, zb, dinv, b3.reshape(1, D), batp)
    return mean.reshape(-1)
